# Initial kernel scaffold; baseline (speedup 1.0000x reference)
import jax
import jax.numpy as jnp
from jax.experimental import pallas as pl


def kernel(x, w0, bias_table, emb_table, field_inter_weights):
    raise NotImplementedError("write your pallas kernel here")


# speedup vs baseline: 15.9654x; 15.9654x over previous
import functools

import jax
import jax.numpy as jnp
from jax import lax
from jax.experimental import pallas as pl
from jax.experimental.pallas import tpu as pltpu
from jax.experimental.pallas import tpu_sc as plsc

_NC = 2
_NS = 16
_NW = _NC * _NS
_L = 16

_IDX_ROW = 128
_BURST = 8


def _sc_gather(x_flat2d, emb_table, bias_table, *, rows_total, emb_dim):
    n_idx_rows = rows_total // _IDX_ROW
    rows_per_w = n_idx_rows // _NW
    bursts = rows_per_w // _BURST

    mesh = plsc.VectorSubcoreMesh(
        core_axis_name="c", subcore_axis_name="s",
        num_cores=_NC, num_subcores=_NS)

    @functools.partial(
        pl.kernel,
        out_type=(
            jax.ShapeDtypeStruct((rows_total, emb_dim), jnp.float32),
            jax.ShapeDtypeStruct((rows_total,), jnp.float32),
        ),
        mesh=mesh,
        scratch_types=[
            pltpu.VMEM((rows_per_w, _IDX_ROW), jnp.int32),
            pltpu.VMEM((_BURST * _IDX_ROW, emb_dim), jnp.float32),
            pltpu.VMEM((_BURST * _IDX_ROW,), jnp.float32),
            pltpu.SemaphoreType.DMA,
            pltpu.SemaphoreType.DMA,
        ],
        compiler_params=pltpu.CompilerParams(use_tc_tiling_on_sc=False),
    )
    def k(x_hbm, emb_hbm, bias_hbm, xg_hbm, bg_hbm, idx_v, ebuf, bbuf, sem_e, sem_b):
        wid = lax.axis_index("s") * _NC + lax.axis_index("c")
        pltpu.sync_copy(x_hbm.at[pl.ds(wid * rows_per_w, rows_per_w)], idx_v)
        for it in range(bursts):
            waits = []
            for g in range(_BURST):
                irow = idx_v.at[it * _BURST + g]
                waits.append(pltpu.async_copy(
                    emb_hbm.at[irow], ebuf.at[pl.ds(g * _IDX_ROW, _IDX_ROW)], sem_e))
                waits.append(pltpu.async_copy(
                    bias_hbm.at[irow], bbuf.at[pl.ds(g * _IDX_ROW, _IDX_ROW)], sem_b))
            for w in waits:
                w.wait()
            base = (wid * rows_per_w + it * _BURST) * _IDX_ROW
            pltpu.sync_copy(ebuf, xg_hbm.at[pl.ds(base, _BURST * _IDX_ROW)])
            pltpu.sync_copy(bbuf, bg_hbm.at[pl.ds(base, _BURST * _IDX_ROW)])

    return k(x_flat2d, emb_table, bias_table.reshape(-1))


def _tc_combine(xg, bias_rows, m_bf16, w0, *, batch, width, num_factors, block_b):
    grid = batch // block_b

    def body(x_ref, b_ref, m_ref, w0_ref, o_ref):
        xb = x_ref[...]
        y = jnp.dot(xb.astype(jnp.bfloat16), m_ref[...],
                    preferred_element_type=jnp.float32)
        inter = jnp.sum(y * xb, axis=1)
        bsum = jnp.sum(b_ref[...], axis=1)
        o_ref[...] = (inter + bsum + w0_ref[0, 0]).reshape(1, 1, block_b)

    out = pl.pallas_call(
        body,
        grid=(grid,),
        in_specs=[
            pl.BlockSpec((block_b, width), lambda i: (i, 0)),
            pl.BlockSpec((block_b, num_factors), lambda i: (i, 0)),
            pl.BlockSpec((width, width), lambda i: (0, 0)),
            pl.BlockSpec(memory_space=pltpu.SMEM),
        ],
        out_specs=pl.BlockSpec((1, 1, block_b), lambda i: (i, 0, 0)),
        out_shape=jax.ShapeDtypeStruct((grid, 1, block_b), jnp.float32),
    )(xg, bias_rows, m_bf16, w0)
    return out.reshape(batch)


def kernel(x, w0, bias_table, emb_table, field_inter_weights):
    batch, num_factors = x.shape
    emb_dim = emb_table.shape[1]
    width = num_factors * emb_dim
    rows_total = batch * num_factors

    x_flat2d = x.reshape(rows_total // _IDX_ROW, _IDX_ROW).astype(jnp.int32)
    xg, bg = _sc_gather(x_flat2d, emb_table, bias_table,
                        rows_total=rows_total, emb_dim=emb_dim)

    iu = jnp.triu(jnp.ones((num_factors, num_factors), jnp.float32), k=1)
    u = field_inter_weights * iu
    m = jnp.kron(u.T, jnp.eye(emb_dim, dtype=jnp.float32)).astype(jnp.bfloat16)

    out = _tc_combine(
        xg.reshape(batch, width),
        bg.reshape(batch, num_factors),
        m,
        w0.reshape(1, 1),
        batch=batch, width=width, num_factors=num_factors, block_b=1024)
    return out


# speedup vs baseline: 20.8147x; 1.3037x over previous
import functools

import jax
import jax.numpy as jnp
from jax import lax
from jax.experimental import pallas as pl
from jax.experimental.pallas import tpu as pltpu
from jax.experimental.pallas import tpu_sc as plsc

_NC = 2
_NS = 16
_NW = _NC * _NS
_L = 16

_IDX_ROW = 128
_BURST = 8


def _sc_transpose(emb_t3, *, num_rows, emb_dim):
    tiles_per_w = (num_rows // 128) // _NW
    cols_per_w = tiles_per_w * 128
    chunk = 2048
    n_full = cols_per_w // chunk
    rem = cols_per_w - n_full * chunk
    extra_base = cols_per_w * _NW
    n_extra = (num_rows - extra_base) // 128
    tail = num_rows - extra_base - n_extra * 128
    half = emb_dim // 2

    mesh = plsc.VectorSubcoreMesh(
        core_axis_name="c", subcore_axis_name="s",
        num_cores=_NC, num_subcores=_NS)

    @functools.partial(
        pl.kernel,
        out_type=jax.ShapeDtypeStruct((num_rows * emb_dim,), jnp.float32),
        mesh=mesh,
        scratch_types=[
            pltpu.VMEM((emb_dim, chunk), jnp.float32),
            pltpu.VMEM((chunk * emb_dim,), jnp.float32),
        ],
        compiler_params=pltpu.CompilerParams(use_tc_tiling_on_sc=True,
                                             needs_layout_passes=False),
    )
    def k(src_hbm, out_hbm, buf, tbuf):
        wid = lax.axis_index("s") * _NC + lax.axis_index("c")
        lanes = lax.iota(jnp.int32, 16)

        def do_chunk(c0, width):
            c0 = pl.multiple_of(c0, 128)
            pltpu.sync_copy(src_hbm.at[0, :, pl.ds(c0, width)],
                            buf.at[pl.ds(0, half), pl.ds(0, width)])
            pltpu.sync_copy(src_hbm.at[1, :, pl.ds(c0, width)],
                            buf.at[pl.ds(half, half), pl.ds(0, width)])

            @plsc.parallel_loop(0, width, 1, unroll=8)
            def _(c):
                vec = plsc.load_gather(buf, [lanes, jnp.full((16,), c, jnp.int32)])
                tbuf[pl.ds(c * emb_dim, 16)] = vec

            pltpu.sync_copy(tbuf.at[pl.ds(0, width * emb_dim)],
                            out_hbm.at[pl.ds(c0 * emb_dim, width * emb_dim)])

        for it in range(n_full):
            do_chunk(wid * cols_per_w + it * chunk, chunk)
        if rem:
            do_chunk(wid * cols_per_w + n_full * chunk, rem)
        for e in range(n_extra):
            @pl.when(wid == e)
            def _():
                do_chunk(extra_base + e * 128, 128)
        if tail:
            @pl.when(wid == n_extra)
            def _():
                do_chunk(extra_base + n_extra * 128, tail)

    return k(emb_t3)


def _sc_gather(x_flat2d, emb_table, bias_table, *, rows_total, emb_dim):
    n_idx_rows = rows_total // _IDX_ROW
    rows_per_w = n_idx_rows // _NW
    bursts = rows_per_w // _BURST

    mesh = plsc.VectorSubcoreMesh(
        core_axis_name="c", subcore_axis_name="s",
        num_cores=_NC, num_subcores=_NS)

    @functools.partial(
        pl.kernel,
        out_type=(
            jax.ShapeDtypeStruct((rows_total, emb_dim), jnp.float32),
            jax.ShapeDtypeStruct((rows_total,), jnp.float32),
        ),
        mesh=mesh,
        scratch_types=[
            pltpu.VMEM((rows_per_w, _IDX_ROW), jnp.int32),
            pltpu.VMEM((_BURST * _IDX_ROW, emb_dim), jnp.float32),
            pltpu.VMEM((_BURST * _IDX_ROW,), jnp.float32),
            pltpu.SemaphoreType.DMA,
            pltpu.SemaphoreType.DMA,
        ],
        compiler_params=pltpu.CompilerParams(use_tc_tiling_on_sc=False),
    )
    def k(x_hbm, emb_hbm, bias_hbm, xg_hbm, bg_hbm, idx_v, ebuf, bbuf, sem_e, sem_b):
        wid = lax.axis_index("s") * _NC + lax.axis_index("c")
        pltpu.sync_copy(x_hbm.at[pl.ds(wid * rows_per_w, rows_per_w)], idx_v)
        for it in range(bursts):
            waits = []
            for g in range(_BURST):
                irow = idx_v.at[it * _BURST + g]
                waits.append(pltpu.async_copy(
                    emb_hbm.at[irow], ebuf.at[pl.ds(g * _IDX_ROW, _IDX_ROW)], sem_e))
                waits.append(pltpu.async_copy(
                    bias_hbm.at[irow], bbuf.at[pl.ds(g * _IDX_ROW, _IDX_ROW)], sem_b))
            for w in waits:
                w.wait()
            base = (wid * rows_per_w + it * _BURST) * _IDX_ROW
            pltpu.sync_copy(ebuf, xg_hbm.at[pl.ds(base, _BURST * _IDX_ROW)])
            pltpu.sync_copy(bbuf, bg_hbm.at[pl.ds(base, _BURST * _IDX_ROW)])

    return k(x_flat2d, emb_table, bias_table.reshape(-1))


def _tc_combine(xg, bias_rows, m_bf16, w0, *, batch, width, num_factors, block_b):
    grid = batch // block_b

    def body(x_ref, b_ref, m_ref, w0_ref, o_ref):
        xb = x_ref[...]
        y = jnp.dot(xb.astype(jnp.bfloat16), m_ref[...],
                    preferred_element_type=jnp.float32)
        inter = jnp.sum(y * xb, axis=1)
        bsum = jnp.sum(b_ref[...], axis=1)
        o_ref[...] = (inter + bsum + w0_ref[0, 0]).reshape(1, 1, block_b)

    out = pl.pallas_call(
        body,
        grid=(grid,),
        in_specs=[
            pl.BlockSpec((block_b, width), lambda i: (i, 0)),
            pl.BlockSpec((block_b, num_factors), lambda i: (i, 0)),
            pl.BlockSpec((width, width), lambda i: (0, 0)),
            pl.BlockSpec(memory_space=pltpu.SMEM),
        ],
        out_specs=pl.BlockSpec((1, 1, block_b), lambda i: (i, 0, 0)),
        out_shape=jax.ShapeDtypeStruct((grid, 1, block_b), jnp.float32),
    )(xg, bias_rows, m_bf16, w0)
    return out.reshape(batch)


def kernel(x, w0, bias_table, emb_table, field_inter_weights):
    batch, num_factors = x.shape
    emb_dim = emb_table.shape[1]
    width = num_factors * emb_dim
    rows_total = batch * num_factors

    x_flat2d = x.reshape(rows_total // _IDX_ROW, _IDX_ROW).astype(jnp.int32)
    num_rows = emb_table.shape[0]
    emb_t3 = emb_table.T.reshape(2, emb_dim // 2, num_rows)
    emb_lin = _sc_transpose(emb_t3, num_rows=num_rows, emb_dim=emb_dim)
    xg, bg = _sc_gather(x_flat2d, emb_lin.reshape(num_rows, emb_dim), bias_table,
                        rows_total=rows_total, emb_dim=emb_dim)

    iu = jnp.triu(jnp.ones((num_factors, num_factors), jnp.float32), k=1)
    u = field_inter_weights * iu
    m = jnp.kron(u.T, jnp.eye(emb_dim, dtype=jnp.float32)).astype(jnp.bfloat16)

    out = _tc_combine(
        xg.reshape(batch, width),
        bg.reshape(batch, num_factors),
        m,
        w0.reshape(1, 1),
        batch=batch, width=width, num_factors=num_factors, block_b=1024)
    return out


# speedup vs baseline: 23.0269x; 1.1063x over previous
import functools

import jax
import jax.numpy as jnp
from jax import lax
from jax.experimental import pallas as pl
from jax.experimental.pallas import tpu as pltpu
from jax.experimental.pallas import tpu_sc as plsc

_NC = 2
_NS = 16
_NW = _NC * _NS
_L = 16

_IDX_ROW = 128
_BURST = 8


def _sc_transpose(emb_t3, *, num_rows, emb_dim):
    tiles_per_w = (num_rows // 128) // _NW
    cols_per_w = tiles_per_w * 128
    chunk = 2048
    n_full = cols_per_w // chunk
    rem = cols_per_w - n_full * chunk
    extra_base = cols_per_w * _NW
    n_extra = (num_rows - extra_base) // 128
    tail = num_rows - extra_base - n_extra * 128
    half = emb_dim // 2

    mesh = plsc.VectorSubcoreMesh(
        core_axis_name="c", subcore_axis_name="s",
        num_cores=_NC, num_subcores=_NS)

    @functools.partial(
        pl.kernel,
        out_type=jax.ShapeDtypeStruct((num_rows * emb_dim,), jnp.float32),
        mesh=mesh,
        scratch_types=[
            pltpu.VMEM((2, emb_dim, chunk), jnp.float32),
            pltpu.VMEM((2, chunk * emb_dim), jnp.float32),
            pltpu.SemaphoreType.DMA,
            pltpu.SemaphoreType.DMA,
            pltpu.SemaphoreType.DMA,
            pltpu.SemaphoreType.DMA,
        ],
        compiler_params=pltpu.CompilerParams(use_tc_tiling_on_sc=True,
                                             needs_layout_passes=False),
    )
    def k(src_hbm, out_hbm, buf, tbuf, sg0, sg1, sw0, sw1):
        wid = lax.axis_index("s") * _NC + lax.axis_index("c")
        lanes = lax.iota(jnp.int32, 16)
        sgs, sws = (sg0, sg1), (sw0, sw1)

        chunks = [(wid * cols_per_w + it * chunk, chunk) for it in range(n_full)]
        if rem:
            chunks.append((wid * cols_per_w + n_full * chunk, rem))
        n = len(chunks)

        def start_in(i):
            c0, w = chunks[i]
            c0 = pl.multiple_of(c0, 128)
            s = i % 2
            return [
                pltpu.async_copy(src_hbm.at[0, :, pl.ds(c0, w)],
                                 buf.at[s, pl.ds(0, half), pl.ds(0, w)], sgs[s]),
                pltpu.async_copy(src_hbm.at[1, :, pl.ds(c0, w)],
                                 buf.at[s, pl.ds(half, half), pl.ds(0, w)], sgs[s]),
            ]

        def compute(i):
            w = chunks[i][1]
            s = i % 2

            @plsc.parallel_loop(0, w, 1, unroll=8)
            def _(c):
                vec = plsc.load_gather(
                    buf.at[s], [lanes, jnp.full((16,), c, jnp.int32)])
                tbuf[s, pl.ds(c * emb_dim, 16)] = vec

        def start_out(i):
            c0, w = chunks[i]
            c0 = pl.multiple_of(c0, 128)
            s = i % 2
            return [pltpu.async_copy(
                tbuf.at[s, pl.ds(0, w * emb_dim)],
                out_hbm.at[pl.ds(c0 * emb_dim, w * emb_dim)], sws[s])]

        pin = {0: start_in(0)}
        pout = {}
        for i in range(n):
            for h in pin.pop(i):
                h.wait()
            if i + 1 < n:
                pin[i + 1] = start_in(i + 1)
            if i - 2 in pout:
                for h in pout.pop(i - 2):
                    h.wait()
            compute(i)
            pout[i] = start_out(i)
        for hs in pout.values():
            for h in hs:
                h.wait()

        def do_sync(c0, w):
            c0 = pl.multiple_of(c0, 128)
            pltpu.sync_copy(src_hbm.at[0, :, pl.ds(c0, w)],
                            buf.at[0, pl.ds(0, half), pl.ds(0, w)])
            pltpu.sync_copy(src_hbm.at[1, :, pl.ds(c0, w)],
                            buf.at[0, pl.ds(half, half), pl.ds(0, w)])

            @plsc.parallel_loop(0, w, 1, unroll=8)
            def _(c):
                vec = plsc.load_gather(
                    buf.at[0], [lanes, jnp.full((16,), c, jnp.int32)])
                tbuf[0, pl.ds(c * emb_dim, 16)] = vec

            pltpu.sync_copy(tbuf.at[0, pl.ds(0, w * emb_dim)],
                            out_hbm.at[pl.ds(c0 * emb_dim, w * emb_dim)])

        for e in range(n_extra):
            @pl.when(wid == e)
            def _():
                do_sync(extra_base + e * 128, 128)
        if tail:
            @pl.when(wid == n_extra)
            def _():
                do_sync(extra_base + n_extra * 128, tail)

    return k(emb_t3)


def _sc_gather(x_flat2d, emb_table, bias_table, *, rows_total, emb_dim):
    n_idx_rows = rows_total // _IDX_ROW
    rows_per_w = n_idx_rows // _NW
    bursts = rows_per_w // _BURST

    mesh = plsc.VectorSubcoreMesh(
        core_axis_name="c", subcore_axis_name="s",
        num_cores=_NC, num_subcores=_NS)

    @functools.partial(
        pl.kernel,
        out_type=(
            jax.ShapeDtypeStruct((rows_total, emb_dim), jnp.float32),
            jax.ShapeDtypeStruct((rows_total,), jnp.float32),
        ),
        mesh=mesh,
        scratch_types=[
            pltpu.VMEM((rows_per_w, _IDX_ROW), jnp.int32),
            pltpu.VMEM((_BURST * _IDX_ROW, emb_dim), jnp.float32),
            pltpu.VMEM((_BURST * _IDX_ROW,), jnp.float32),
            pltpu.SemaphoreType.DMA,
            pltpu.SemaphoreType.DMA,
        ],
        compiler_params=pltpu.CompilerParams(use_tc_tiling_on_sc=False),
    )
    def k(x_hbm, emb_hbm, bias_hbm, xg_hbm, bg_hbm, idx_v, ebuf, bbuf, sem_e, sem_b):
        wid = lax.axis_index("s") * _NC + lax.axis_index("c")
        pltpu.sync_copy(x_hbm.at[pl.ds(wid * rows_per_w, rows_per_w)], idx_v)
        for it in range(bursts):
            waits = []
            for g in range(_BURST):
                irow = idx_v.at[it * _BURST + g]
                waits.append(pltpu.async_copy(
                    emb_hbm.at[irow], ebuf.at[pl.ds(g * _IDX_ROW, _IDX_ROW)], sem_e))
                waits.append(pltpu.async_copy(
                    bias_hbm.at[irow], bbuf.at[pl.ds(g * _IDX_ROW, _IDX_ROW)], sem_b))
            for w in waits:
                w.wait()
            base = (wid * rows_per_w + it * _BURST) * _IDX_ROW
            pltpu.sync_copy(ebuf, xg_hbm.at[pl.ds(base, _BURST * _IDX_ROW)])
            pltpu.sync_copy(bbuf, bg_hbm.at[pl.ds(base, _BURST * _IDX_ROW)])

    return k(x_flat2d, emb_table, bias_table.reshape(-1))


def _tc_combine(xg, bias_rows, m_bf16, w0, *, batch, width, num_factors, block_b):
    grid = batch // block_b

    def body(x_ref, b_ref, m_ref, w0_ref, o_ref):
        xb = x_ref[...]
        y = jnp.dot(xb.astype(jnp.bfloat16), m_ref[...],
                    preferred_element_type=jnp.float32)
        inter = jnp.sum(y * xb, axis=1)
        bsum = jnp.sum(b_ref[...], axis=1)
        o_ref[...] = (inter + bsum + w0_ref[0, 0]).reshape(1, 1, block_b)

    out = pl.pallas_call(
        body,
        grid=(grid,),
        in_specs=[
            pl.BlockSpec((block_b, width), lambda i: (i, 0)),
            pl.BlockSpec((block_b, num_factors), lambda i: (i, 0)),
            pl.BlockSpec((width, width), lambda i: (0, 0)),
            pl.BlockSpec(memory_space=pltpu.SMEM),
        ],
        out_specs=pl.BlockSpec((1, 1, block_b), lambda i: (i, 0, 0)),
        out_shape=jax.ShapeDtypeStruct((grid, 1, block_b), jnp.float32),
    )(xg, bias_rows, m_bf16, w0)
    return out.reshape(batch)


def kernel(x, w0, bias_table, emb_table, field_inter_weights):
    batch, num_factors = x.shape
    emb_dim = emb_table.shape[1]
    width = num_factors * emb_dim
    rows_total = batch * num_factors

    x_flat2d = x.reshape(rows_total // _IDX_ROW, _IDX_ROW).astype(jnp.int32)
    num_rows = emb_table.shape[0]
    emb_t3 = emb_table.T.reshape(2, emb_dim // 2, num_rows)
    emb_lin = _sc_transpose(emb_t3, num_rows=num_rows, emb_dim=emb_dim)
    xg, bg = _sc_gather(x_flat2d, emb_lin.reshape(num_rows, emb_dim), bias_table,
                        rows_total=rows_total, emb_dim=emb_dim)

    iu = jnp.triu(jnp.ones((num_factors, num_factors), jnp.float32), k=1)
    u = field_inter_weights * iu
    m = jnp.kron(u.T, jnp.eye(emb_dim, dtype=jnp.float32)).astype(jnp.bfloat16)

    out = _tc_combine(
        xg.reshape(batch, width),
        bg.reshape(batch, num_factors),
        m,
        w0.reshape(1, 1),
        batch=batch, width=width, num_factors=num_factors, block_b=1024)
    return out


# speedup vs baseline: 24.6401x; 1.0701x over previous
import functools

import jax
import jax.numpy as jnp
from jax import lax
from jax.experimental import pallas as pl
from jax.experimental.pallas import tpu as pltpu
from jax.experimental.pallas import tpu_sc as plsc

_NC = 2
_NS = 16
_NW = _NC * _NS
_L = 16

_IDX_ROW = 128
_BURST = 8


def _sc_transpose(emb_t3, *, num_rows, emb_dim):
    tiles_per_w = (num_rows // 128) // _NW
    cols_per_w = tiles_per_w * 128
    chunk = 1536
    n_full = cols_per_w // chunk
    rem = cols_per_w - n_full * chunk
    extra_base = cols_per_w * _NW
    n_extra = (num_rows - extra_base) // 128
    tail = num_rows - extra_base - n_extra * 128
    half = emb_dim // 2
    cpad = chunk + 1

    mesh = plsc.VectorSubcoreMesh(
        core_axis_name="c", subcore_axis_name="s",
        num_cores=_NC, num_subcores=_NS)

    @functools.partial(
        pl.kernel,
        out_type=jax.ShapeDtypeStruct((num_rows * emb_dim,), jnp.float32),
        mesh=mesh,
        scratch_types=[
            pltpu.VMEM((2, emb_dim, cpad), jnp.float32),
            pltpu.VMEM((2, chunk * emb_dim), jnp.float32),
            pltpu.SemaphoreType.DMA,
            pltpu.SemaphoreType.DMA,
            pltpu.SemaphoreType.DMA,
            pltpu.SemaphoreType.DMA,
        ],
        compiler_params=pltpu.CompilerParams(use_tc_tiling_on_sc=True,
                                             needs_layout_passes=False),
    )
    def k(src_hbm, out_hbm, buf, tbuf, sg0, sg1, sw0, sw1):
        wid = lax.axis_index("s") * _NC + lax.axis_index("c")
        lanes = lax.iota(jnp.int32, 16)
        sgs, sws = (sg0, sg1), (sw0, sw1)

        chunks = [(wid * cols_per_w + it * chunk, chunk) for it in range(n_full)]
        if rem:
            chunks.append((wid * cols_per_w + n_full * chunk, rem))
        n = len(chunks)

        def start_in(i):
            c0, w = chunks[i]
            c0 = pl.multiple_of(c0, 128)
            s = i % 2
            return [
                pltpu.async_copy(src_hbm.at[0, :, pl.ds(c0, w)],
                                 buf.at[s, pl.ds(0, half), pl.ds(0, w)], sgs[s]),
                pltpu.async_copy(src_hbm.at[1, :, pl.ds(c0, w)],
                                 buf.at[s, pl.ds(half, half), pl.ds(0, w)], sgs[s]),
            ]

        def compute(i):
            w = chunks[i][1]
            s = i % 2

            @plsc.parallel_loop(0, w, 1, unroll=8,
                                carry=jnp.zeros((16,), jnp.int32))
            def _(c, vcol):
                vec = plsc.load_gather(buf.at[s], [lanes, vcol])
                tbuf[s, pl.ds(c * emb_dim, 16)] = vec
                return vcol + 1

        def start_out(i):
            c0, w = chunks[i]
            c0 = pl.multiple_of(c0, 128)
            s = i % 2
            return [pltpu.async_copy(
                tbuf.at[s, pl.ds(0, w * emb_dim)],
                out_hbm.at[pl.ds(c0 * emb_dim, w * emb_dim)], sws[s])]

        pin = {0: start_in(0)}
        pout = {}
        for i in range(n):
            for h in pin.pop(i):
                h.wait()
            if i + 1 < n:
                pin[i + 1] = start_in(i + 1)
            if i - 2 in pout:
                for h in pout.pop(i - 2):
                    h.wait()
            compute(i)
            pout[i] = start_out(i)
        for hs in pout.values():
            for h in hs:
                h.wait()

        def do_sync(c0, w):
            c0 = pl.multiple_of(c0, 128)
            pltpu.sync_copy(src_hbm.at[0, :, pl.ds(c0, w)],
                            buf.at[0, pl.ds(0, half), pl.ds(0, w)])
            pltpu.sync_copy(src_hbm.at[1, :, pl.ds(c0, w)],
                            buf.at[0, pl.ds(half, half), pl.ds(0, w)])

            @plsc.parallel_loop(0, w, 1, unroll=8,
                                carry=jnp.zeros((16,), jnp.int32))
            def _(c, vcol):
                vec = plsc.load_gather(buf.at[0], [lanes, vcol])
                tbuf[0, pl.ds(c * emb_dim, 16)] = vec
                return vcol + 1

            pltpu.sync_copy(tbuf.at[0, pl.ds(0, w * emb_dim)],
                            out_hbm.at[pl.ds(c0 * emb_dim, w * emb_dim)])

        for e in range(n_extra):
            @pl.when(wid == e)
            def _():
                do_sync(extra_base + e * 128, 128)
        if tail:
            @pl.when(wid == n_extra)
            def _():
                do_sync(extra_base + n_extra * 128, tail)

    return k(emb_t3)


def _sc_gather(x_flat2d, emb_table, bias_table, *, rows_total, emb_dim):
    n_idx_rows = rows_total // _IDX_ROW
    rows_per_w = n_idx_rows // _NW
    bursts = rows_per_w // _BURST

    mesh = plsc.VectorSubcoreMesh(
        core_axis_name="c", subcore_axis_name="s",
        num_cores=_NC, num_subcores=_NS)

    @functools.partial(
        pl.kernel,
        out_type=(
            jax.ShapeDtypeStruct((rows_total, emb_dim), jnp.float32),
            jax.ShapeDtypeStruct((rows_total,), jnp.float32),
        ),
        mesh=mesh,
        scratch_types=[
            pltpu.VMEM((rows_per_w, _IDX_ROW), jnp.int32),
            pltpu.VMEM((_BURST * _IDX_ROW, emb_dim), jnp.float32),
            pltpu.VMEM((_BURST * _IDX_ROW,), jnp.float32),
            pltpu.SemaphoreType.DMA,
            pltpu.SemaphoreType.DMA,
        ],
        compiler_params=pltpu.CompilerParams(use_tc_tiling_on_sc=False),
    )
    def k(x_hbm, emb_hbm, bias_hbm, xg_hbm, bg_hbm, idx_v, ebuf, bbuf, sem_e, sem_b):
        wid = lax.axis_index("s") * _NC + lax.axis_index("c")
        pltpu.sync_copy(x_hbm.at[pl.ds(wid * rows_per_w, rows_per_w)], idx_v)
        for it in range(bursts):
            waits = []
            for g in range(_BURST):
                irow = idx_v.at[it * _BURST + g]
                waits.append(pltpu.async_copy(
                    emb_hbm.at[irow], ebuf.at[pl.ds(g * _IDX_ROW, _IDX_ROW)], sem_e))
                waits.append(pltpu.async_copy(
                    bias_hbm.at[irow], bbuf.at[pl.ds(g * _IDX_ROW, _IDX_ROW)], sem_b))
            for w in waits:
                w.wait()
            base = (wid * rows_per_w + it * _BURST) * _IDX_ROW
            pltpu.sync_copy(ebuf, xg_hbm.at[pl.ds(base, _BURST * _IDX_ROW)])
            pltpu.sync_copy(bbuf, bg_hbm.at[pl.ds(base, _BURST * _IDX_ROW)])

    return k(x_flat2d, emb_table, bias_table.reshape(-1))


def _tc_combine(xg, bias_rows, m_bf16, w0, *, batch, width, num_factors, block_b):
    grid = batch // block_b

    def body(x_ref, b_ref, m_ref, w0_ref, o_ref):
        xb = x_ref[...]
        y = jnp.dot(xb.astype(jnp.bfloat16), m_ref[...],
                    preferred_element_type=jnp.float32)
        inter = jnp.sum(y * xb, axis=1)
        bsum = jnp.sum(b_ref[...], axis=1)
        o_ref[...] = (inter + bsum + w0_ref[0, 0]).reshape(1, 1, block_b)

    out = pl.pallas_call(
        body,
        grid=(grid,),
        in_specs=[
            pl.BlockSpec((block_b, width), lambda i: (i, 0)),
            pl.BlockSpec((block_b, num_factors), lambda i: (i, 0)),
            pl.BlockSpec((width, width), lambda i: (0, 0)),
            pl.BlockSpec(memory_space=pltpu.SMEM),
        ],
        out_specs=pl.BlockSpec((1, 1, block_b), lambda i: (i, 0, 0)),
        out_shape=jax.ShapeDtypeStruct((grid, 1, block_b), jnp.float32),
    )(xg, bias_rows, m_bf16, w0)
    return out.reshape(batch)


def kernel(x, w0, bias_table, emb_table, field_inter_weights):
    batch, num_factors = x.shape
    emb_dim = emb_table.shape[1]
    width = num_factors * emb_dim
    rows_total = batch * num_factors

    x_flat2d = x.reshape(rows_total // _IDX_ROW, _IDX_ROW).astype(jnp.int32)
    num_rows = emb_table.shape[0]
    emb_t3 = emb_table.T.reshape(2, emb_dim // 2, num_rows)
    emb_lin = _sc_transpose(emb_t3, num_rows=num_rows, emb_dim=emb_dim)
    xg, bg = _sc_gather(x_flat2d, emb_lin.reshape(num_rows, emb_dim), bias_table,
                        rows_total=rows_total, emb_dim=emb_dim)

    iu = jnp.triu(jnp.ones((num_factors, num_factors), jnp.float32), k=1)
    u = field_inter_weights * iu
    m = jnp.kron(u.T, jnp.eye(emb_dim, dtype=jnp.float32)).astype(jnp.bfloat16)

    out = _tc_combine(
        xg.reshape(batch, width),
        bg.reshape(batch, num_factors),
        m,
        w0.reshape(1, 1),
        batch=batch, width=width, num_factors=num_factors, block_b=1024)
    return out
